# baseline (speedup 1.0000x reference)
import functools

import jax
import jax.numpy as jnp
from jax import lax
from jax.experimental import pallas as pl
from jax.experimental.pallas import tpu as pltpu
from jax.experimental.pallas import tpu_sc as plsc

_NC = 2
_NS = 16
_NW = _NC * _NS


@functools.lru_cache(maxsize=None)
def _make_lookup(V, D, B):
  assert B % (_NW * 8) == 0
  b_per_w = B // _NW
  mesh = plsc.VectorSubcoreMesh(core_axis_name="c", subcore_axis_name="s")

  @functools.partial(
      pl.kernel,
      mesh=mesh,
      out_type=jax.ShapeDtypeStruct((B, D), jnp.float32),
      scratch_types=[
          pltpu.VMEM((b_per_w,), jnp.int32),
          pltpu.VMEM((b_per_w, D), jnp.float32),
          pltpu.SemaphoreType.DMA,
      ],
      compiler_params=pltpu.CompilerParams(use_tc_tiling_on_sc=False),
  )
  def lookup(table_hbm, idx_hbm, out_hbm, idx_v, rows_v, sem):
    wid = lax.axis_index("s") * _NC + lax.axis_index("c")
    base = wid * b_per_w
    pltpu.sync_copy(idx_hbm.at[pl.ds(base, b_per_w)], idx_v)
    pltpu.async_copy(table_hbm.at[idx_v], rows_v, sem).wait()
    pltpu.sync_copy(rows_v, out_hbm.at[pl.ds(base, b_per_w)])

  return lookup


def kernel(sentence, table):
  (B,) = sentence.shape
  V, D = table.shape
  idx = sentence.astype(jnp.int32)
  return _make_lookup(V, D, B)(table, idx)


# speedup vs baseline: 1.3270x; 1.3270x over previous
import functools

import jax
import jax.numpy as jnp
from jax import lax
from jax.experimental import pallas as pl
from jax.experimental.pallas import tpu as pltpu
from jax.experimental.pallas import tpu_sc as plsc

_NC = 2
_NS = 16
_NW = _NC * _NS
_LANES = 128


@functools.lru_cache(maxsize=None)
def _make_lookup(V, D, B):
  assert D <= _LANES and B % (_NW * 8) == 0
  b_per_w = B // _NW
  mesh = plsc.VectorSubcoreMesh(core_axis_name="c", subcore_axis_name="s")

  @functools.partial(
      pl.kernel,
      mesh=mesh,
      out_type=jax.ShapeDtypeStruct((B, _LANES), jnp.float32),
      scratch_types=[
          pltpu.VMEM((b_per_w,), jnp.int32),
          pltpu.VMEM((b_per_w, D), jnp.float32),
          pltpu.VMEM_SHARED((V, D), jnp.float32),
          pltpu.SemaphoreType.DMA,
      ],
      compiler_params=pltpu.CompilerParams(use_tc_tiling_on_sc=False),
  )
  def lookup(table_hbm, idx_hbm, out_hbm, idx_v, rows_v, table_s, sem):
    sid = lax.axis_index("s")
    wid = sid * _NC + lax.axis_index("c")
    base = wid * b_per_w
    @pl.when(sid == 0)
    def _():
      pltpu.sync_copy(table_hbm, table_s)
    pltpu.sync_copy(idx_hbm.at[pl.ds(base, b_per_w)], idx_v)
    plsc.subcore_barrier()
    pltpu.async_copy(table_s.at[idx_v], rows_v, sem).wait()
    pltpu.sync_copy(rows_v, out_hbm.at[pl.ds(base, b_per_w), pl.ds(0, D)])

  return lookup


def kernel(sentence, table):
  (B,) = sentence.shape
  V, D = table.shape
  idx = sentence.astype(jnp.int32)
  out128 = _make_lookup(V, D, B)(table, idx)
  return out128[:, :D]


# speedup vs baseline: 1.3283x; 1.0010x over previous
import functools

import jax
import jax.numpy as jnp
from jax import lax
from jax.experimental import pallas as pl
from jax.experimental.pallas import tpu as pltpu
from jax.experimental.pallas import tpu_sc as plsc

_NC = 2
_NS = 16
_NW = _NC * _NS
_LANES = 128


@functools.lru_cache(maxsize=None)
def _make_lookup(V, D, B):
  assert D <= _LANES and B % (_NW * 8) == 0
  b_per_w = B // _NW
  mesh = plsc.VectorSubcoreMesh(core_axis_name="c", subcore_axis_name="s")

  @functools.partial(
      pl.kernel,
      mesh=mesh,
      out_type=jax.ShapeDtypeStruct((B, _LANES), jnp.float32),
      scratch_types=[
          pltpu.VMEM((b_per_w,), jnp.int32),
          pltpu.VMEM((b_per_w, D), jnp.float32),
          pltpu.VMEM_SHARED((V, D), jnp.float32),
          pltpu.SemaphoreType.DMA,
          pltpu.SemaphoreType.DMA,
          pltpu.SemaphoreType.DMA,
      ],
      compiler_params=pltpu.CompilerParams(use_tc_tiling_on_sc=False),
  )
  def lookup(table_hbm, idx_hbm, out_hbm, idx_v, rows_v, table_s,
             gsem0, gsem1, ssem):
    half = b_per_w // 2
    sid = lax.axis_index("s")
    wid = sid * _NC + lax.axis_index("c")
    base = wid * b_per_w
    @pl.when(sid == 0)
    def _():
      pltpu.sync_copy(table_hbm, table_s)
    pltpu.sync_copy(idx_hbm.at[pl.ds(base, b_per_w)], idx_v)
    plsc.subcore_barrier()
    g0 = pltpu.async_copy(table_s.at[idx_v.at[pl.ds(0, half)]],
                          rows_v.at[pl.ds(0, half)], gsem0)
    g1 = pltpu.async_copy(table_s.at[idx_v.at[pl.ds(half, half)]],
                          rows_v.at[pl.ds(half, half)], gsem1)
    g0.wait()
    s0 = pltpu.async_copy(
        rows_v.at[pl.ds(0, half)],
        out_hbm.at[pl.ds(base, half), pl.ds(0, D)], ssem)
    g1.wait()
    s1 = pltpu.async_copy(
        rows_v.at[pl.ds(half, half)],
        out_hbm.at[pl.ds(base + half, half), pl.ds(0, D)], ssem)
    s0.wait()
    s1.wait()

  return lookup


def kernel(sentence, table):
  (B,) = sentence.shape
  V, D = table.shape
  idx = sentence.astype(jnp.int32)
  out128 = _make_lookup(V, D, B)(table, idx)
  return out128[:, :D]


# speedup vs baseline: 1.3317x; 1.0026x over previous
import functools

import jax
import jax.numpy as jnp
from jax import lax
from jax.experimental import pallas as pl
from jax.experimental.pallas import tpu as pltpu
from jax.experimental.pallas import tpu_sc as plsc

_NC = 2
_NS = 16
_NW = _NC * _NS
_LANES = 128


@functools.lru_cache(maxsize=None)
def _make_lookup(V, D, B):
  assert D <= _LANES and B % (_NW * 8) == 0
  b_per_w = B // _NW
  mesh = plsc.VectorSubcoreMesh(core_axis_name="c", subcore_axis_name="s")

  @functools.partial(
      pl.kernel,
      mesh=mesh,
      out_type=jax.ShapeDtypeStruct((B, _LANES), jnp.float32),
      scratch_types=[
          pltpu.VMEM((b_per_w,), jnp.int32),
          pltpu.VMEM((b_per_w, D), jnp.float32),
          pltpu.VMEM_SHARED((V, D), jnp.float32),
          pltpu.SemaphoreType.DMA,
      ],
      compiler_params=pltpu.CompilerParams(use_tc_tiling_on_sc=False),
  )
  def lookup(table_hbm, idx_hbm, out_hbm, idx_v, rows_v, table_s, sem):
    sid = lax.axis_index("s")
    wid = sid * _NC + lax.axis_index("c")
    base = wid * b_per_w
    @pl.when(sid == 0)
    def _():
      pltpu.sync_copy(table_hbm, table_s)
    pltpu.sync_copy(idx_hbm.at[pl.ds(base, b_per_w)], idx_v)
    plsc.subcore_barrier()
    pltpu.async_copy(table_s.at[idx_v], rows_v, sem).wait()
    pltpu.sync_copy(rows_v, out_hbm.at[pl.ds(base, b_per_w), pl.ds(0, D)])

  return lookup


def kernel(sentence, table):
  (B,) = sentence.shape
  V, D = table.shape
  idx = sentence.astype(jnp.int32)
  out128 = _make_lookup(V, D, B)(table, idx)
  return out128[:, :D]
